# baseline (speedup 1.0000x reference)
import functools

import jax
import jax.numpy as jnp
from jax import lax
from jax.experimental import pallas as pl
from jax.experimental.pallas import tpu as pltpu
from jax.experimental.pallas import tpu_sc as plsc

HIDDEN = 768
EPS = 1e-9
LANES = 16
NVEC = HIDDEN // LANES
K = 64


def _lane_sum(x):
    dnums = lax.GatherDimensionNumbers(
        offset_dims=(), collapsed_slice_dims=(0,), start_index_map=(0,))
    for k in (1, 2, 4, 8):
        perm = (lax.iota(jnp.int32, LANES) ^ k).reshape(LANES, 1)
        x = x + lax.gather(x, perm, dnums, (1,),
                           mode=lax.GatherScatterMode.PROMISE_IN_BOUNDS)
    return x


def _rsqrt_vec(v):
    i = lax.bitcast_convert_type(v, jnp.int32)
    i = 0x5F3759DF - (i >> 1)
    y = lax.bitcast_convert_type(i, jnp.float32)
    for _ in range(3):
        y = y * (1.5 - 0.5 * v * y * y)
    return y


def _make_sc_kernel(n_rows):
    info = plsc.get_sparse_core_info()
    nc, ns = info.num_cores, info.num_subcores
    nw = nc * ns
    rows_per_tile = n_rows // nw
    chunks = rows_per_tile // K
    mesh = plsc.VectorSubcoreMesh(core_axis_name="c", subcore_axis_name="s")

    @functools.partial(
        pl.kernel,
        mesh=mesh,
        out_type=jax.ShapeDtypeStruct((n_rows, HIDDEN), jnp.float32),
        scratch_types=[
            pltpu.VMEM((K,), jnp.int32),
            pltpu.VMEM((K, HIDDEN), jnp.float32),
            pltpu.VMEM((HIDDEN,), jnp.float32),
            pltpu.VMEM((HIDDEN,), jnp.float32),
            pltpu.SemaphoreType.DMA,
        ],
    )
    def emb_ln(ids_hbm, table_hbm, gamma_hbm, beta_hbm, out_hbm,
               idx_v, rows_v, gamma_v, beta_v, sem):
        wid = lax.axis_index("s") * nc + lax.axis_index("c")
        base = wid * rows_per_tile
        pltpu.sync_copy(gamma_hbm, gamma_v)
        pltpu.sync_copy(beta_hbm, beta_v)

        def chunk_body(c, carry):
            rb = base + c * K
            pltpu.sync_copy(ids_hbm.at[pl.ds(rb, K)], idx_v)
            pltpu.async_copy(table_hbm.at[idx_v], rows_v, sem).wait()

            def row_body(r, carry2):
                x0 = rows_v[r, pl.ds(0, LANES)]
                acc = x0
                acc2 = x0 * x0
                for j in range(1, NVEC):
                    x = rows_v[r, pl.ds(j * LANES, LANES)]
                    acc = acc + x
                    acc2 = acc2 + x * x
                meanv = _lane_sum(acc) * (1.0 / HIDDEN)
                varv = _lane_sum(acc2) * (1.0 / HIDDEN) - meanv * meanv
                inv = _rsqrt_vec(varv + EPS)
                for j in range(NVEC):
                    sl = pl.ds(j * LANES, LANES)
                    x = rows_v[r, sl]
                    rows_v[r, sl] = ((x - meanv) * inv) * gamma_v[sl] + beta_v[sl]
                return carry2

            lax.fori_loop(0, K, row_body, 0)
            pltpu.sync_copy(rows_v, out_hbm.at[pl.ds(rb, K)])
            return carry

        lax.fori_loop(0, chunks, chunk_body, 0)

    return emb_ln


def kernel(input_ids, word_embeddings, ln_gamma, ln_beta):
    b, s = input_ids.shape
    ids = input_ids.reshape(-1).astype(jnp.int32)
    sc = _make_sc_kernel(b * s)
    out = sc(ids, word_embeddings, ln_gamma, ln_beta)
    return out.reshape(b, s, HIDDEN)


# speedup vs baseline: 3.1093x; 3.1093x over previous
import functools

import jax
import jax.numpy as jnp
from jax import lax
from jax.experimental import pallas as pl
from jax.experimental.pallas import tpu as pltpu
from jax.experimental.pallas import tpu_sc as plsc

HIDDEN = 768
EPS = 1e-9
LANES = 16
NVEC = HIDDEN // LANES
K = 32
RPG = 8
GROUPS = K // RPG


def _lane_sum(x):
    dnums = lax.GatherDimensionNumbers(
        offset_dims=(), collapsed_slice_dims=(0,), start_index_map=(0,))
    for k in (1, 2, 4, 8):
        perm = (lax.iota(jnp.int32, LANES) ^ k).reshape(LANES, 1)
        x = x + lax.gather(x, perm, dnums, (1,),
                           mode=lax.GatherScatterMode.PROMISE_IN_BOUNDS)
    return x


def _rsqrt_vec(v):
    i = lax.bitcast_convert_type(v, jnp.int32)
    i = 0x5F3759DF - (i >> 1)
    y = lax.bitcast_convert_type(i, jnp.float32)
    for _ in range(3):
        y = y * (1.5 - 0.5 * v * y * y)
    return y


def _make_sc_kernel(n_rows):
    info = plsc.get_sparse_core_info()
    nc, ns = info.num_cores, info.num_subcores
    nw = nc * ns
    rows_per_tile = n_rows // nw
    chunks = rows_per_tile // K
    mesh = plsc.VectorSubcoreMesh(core_axis_name="c", subcore_axis_name="s")

    @functools.partial(
        pl.kernel,
        mesh=mesh,
        out_type=jax.ShapeDtypeStruct((n_rows, HIDDEN), jnp.float32),
        scratch_types=[
            pltpu.VMEM((rows_per_tile,), jnp.int32),
            pltpu.VMEM((2, K, HIDDEN), jnp.float32),
            pltpu.VMEM((2, K, HIDDEN), jnp.float32),
            pltpu.VMEM((HIDDEN,), jnp.float32),
            pltpu.VMEM((HIDDEN,), jnp.float32),
            pltpu.SemaphoreType.DMA,
            pltpu.SemaphoreType.DMA,
            pltpu.SemaphoreType.DMA,
            pltpu.SemaphoreType.DMA,
        ],
    )
    def emb_ln(ids_hbm, table_hbm, gamma_hbm, beta_hbm, out_hbm,
               idx_all, inbuf, outbuf, gamma_v, beta_v, g0, g1, s0, s1):
        gsem = (g0, g1)
        ssem = (s0, s1)
        wid = lax.axis_index("s") * nc + lax.axis_index("c")
        base = wid * rows_per_tile
        pltpu.sync_copy(ids_hbm.at[pl.ds(base, rows_per_tile)], idx_all)
        pltpu.sync_copy(gamma_hbm, gamma_v)
        pltpu.sync_copy(beta_hbm, beta_v)

        def gather_start(j, b):
            pltpu.async_copy(
                table_hbm.at[idx_all.at[pl.ds(j * K, K)]], inbuf.at[b], gsem[b])

        def gather_wait(j, b):
            pltpu.make_async_copy(
                table_hbm.at[idx_all.at[pl.ds(j * K, K)]], inbuf.at[b],
                gsem[b]).wait()

        def store_start(j, b):
            pltpu.async_copy(
                outbuf.at[b], out_hbm.at[pl.ds(base + j * K, K)], ssem[b])

        def store_wait(j, b):
            pltpu.make_async_copy(
                outbuf.at[b], out_hbm.at[pl.ds(base + j * K, K)],
                ssem[b]).wait()

        def compute_chunk(b):
            for g in range(GROUPS):
                r0 = g * RPG

                def p1_body(j, carry):
                    accs = list(carry[:RPG])
                    acc2s = list(carry[RPG:])
                    col = pl.ds(j * LANES, LANES)
                    for r in range(RPG):
                        x = inbuf[b, r0 + r, col]
                        accs[r] = accs[r] + x
                        acc2s[r] = acc2s[r] + x * x
                    return tuple(accs) + tuple(acc2s)

                zero = jnp.zeros((LANES,), jnp.float32)
                carry = lax.fori_loop(0, NVEC, p1_body, (zero,) * (2 * RPG))
                invs, ms = [], []
                for r in range(RPG):
                    meanv = _lane_sum(carry[r]) * (1.0 / HIDDEN)
                    varv = (_lane_sum(carry[RPG + r]) * (1.0 / HIDDEN)
                            - meanv * meanv)
                    inv = _rsqrt_vec(varv + EPS)
                    invs.append(inv)
                    ms.append(meanv * inv)

                def p2_body(j, carry2):
                    col = pl.ds(j * LANES, LANES)
                    gj = gamma_v[col]
                    bj = beta_v[col]
                    for r in range(RPG):
                        x = inbuf[b, r0 + r, col]
                        y = x * invs[r] - ms[r]
                        outbuf[b, r0 + r, col] = y * gj + bj
                    return carry2

                lax.fori_loop(0, NVEC, p2_body, 0)

        gather_start(0, 0)

        def outer(o, carry):
            for b in range(2):
                j = 2 * o + b

                @pl.when(j + 1 < chunks)
                def _():
                    gather_start(j + 1, 1 - b)

                gather_wait(j, b)

                @pl.when(j >= 2)
                def _():
                    store_wait(j - 2, b)

                compute_chunk(b)
                store_start(j, b)
            return carry

        lax.fori_loop(0, chunks // 2, outer, 0)
        store_wait(chunks - 2, 0)
        store_wait(chunks - 1, 1)

    return emb_ln


def kernel(input_ids, word_embeddings, ln_gamma, ln_beta):
    b, s = input_ids.shape
    ids = input_ids.reshape(-1).astype(jnp.int32)
    sc = _make_sc_kernel(b * s)
    out = sc(ids, word_embeddings, ln_gamma, ln_beta)
    return out.reshape(b, s, HIDDEN)
